# baseline (speedup 1.0000x reference)
import functools

import jax
import jax.numpy as jnp
from jax.experimental import pallas as pl
from jax.experimental.pallas import tpu as pltpu

_NUM_EMBEDDINGS = 1024
_EMBEDDING_DIM = 64
_COMMITMENT_COST = 0.25


def _vq_block_kernel(x_ref, w_ref, zsq_ref, wsq_ref, zraw_ref,
                     zq_ref, loss_ref, uniq_ref,
                     pres_ref, acc_ref, *, n_tokens, t_blk):
    i = pl.program_id(0)
    n_steps = pl.num_programs(0)

    @pl.when(i == 0)
    def _init():
        pres_ref[...] = jnp.zeros_like(pres_ref)
        acc_ref[0, 0] = jnp.float32(0.0)

    a = x_ref[...]
    w = w_ref[...]

    m = jax.lax.dot_general(a, w, (((1,), (1,)), ((), ())),
                            preferred_element_type=jnp.float32)
    d = (zsq_ref[...] - 2.0 * m) + wsq_ref[...]

    mind = jnp.min(d, axis=1, keepdims=True)
    iota = jax.lax.broadcasted_iota(jnp.int32, (t_blk, _NUM_EMBEDDINGS), 1)
    idx = jnp.min(jnp.where(d == mind, iota, _NUM_EMBEDDINGS),
                  axis=1).astype(jnp.int32)

    oh = (iota == idx[:, None]).astype(jnp.float32)
    zq_t = jax.lax.dot_general(w, oh, (((0,), (1,)), ((), ())),
                               preferred_element_type=jnp.float32)
    zr = zraw_ref[0]
    zq_ref[0] = zr + (zq_t - zr)

    pres_ref[...] = jnp.maximum(pres_ref[...], jnp.max(oh, axis=0,
                                                       keepdims=True))
    acc_ref[0, 0] += jnp.sum(mind[:, 0])

    @pl.when(i == n_steps - 1)
    def _finish():
        total = acc_ref[0, 0]
        mean = total / jnp.float32(n_tokens * _EMBEDDING_DIM)
        loss_ref[0, 0] = mean + jnp.float32(_COMMITMENT_COST) * mean
        uniq_ref[0, 0] = jnp.sum(pres_ref[...]).astype(jnp.int32)


@jax.jit
def kernel(z, W):
    b, c, dd, hh, ww = z.shape
    n_tokens = b * dd * hh * ww
    spatial = dd * hh * ww
    t_blk = 512
    blocks_per_batch = spatial // t_blk
    flat_z = jnp.transpose(z, (0, 2, 3, 4, 1)).reshape(n_tokens, c)
    zsq = jnp.sum(flat_z ** 2, axis=1, keepdims=True)
    wsq = jnp.sum(W ** 2, axis=1).reshape(1, _NUM_EMBEDDINGS)
    z_raw = z.reshape(b, c, spatial)

    grid = (n_tokens // t_blk,)
    zq_raw, loss, uniq = pl.pallas_call(
        functools.partial(_vq_block_kernel, n_tokens=n_tokens, t_blk=t_blk),
        grid=grid,
        in_specs=[
            pl.BlockSpec((t_blk, c), lambda i: (i, 0)),
            pl.BlockSpec((_NUM_EMBEDDINGS, c), lambda i: (0, 0)),
            pl.BlockSpec((t_blk, 1), lambda i: (i, 0)),
            pl.BlockSpec((1, _NUM_EMBEDDINGS), lambda i: (0, 0)),
            pl.BlockSpec((1, c, t_blk),
                         lambda i: (i // blocks_per_batch, 0,
                                    i % blocks_per_batch)),
        ],
        out_specs=[
            pl.BlockSpec((1, c, t_blk),
                         lambda i: (i // blocks_per_batch, 0,
                                    i % blocks_per_batch)),
            pl.BlockSpec((1, 1), lambda i: (0, 0), memory_space=pltpu.SMEM),
            pl.BlockSpec((1, 1), lambda i: (0, 0), memory_space=pltpu.SMEM),
        ],
        out_shape=[
            jax.ShapeDtypeStruct((b, c, spatial), jnp.float32),
            jax.ShapeDtypeStruct((1, 1), jnp.float32),
            jax.ShapeDtypeStruct((1, 1), jnp.int32),
        ],
        scratch_shapes=[
            pltpu.VMEM((1, _NUM_EMBEDDINGS), jnp.float32),
            pltpu.SMEM((1, 1), jnp.float32),
        ],
    )(flat_z, W, zsq, wsq, z_raw)

    return (zq_raw.reshape(b, c, dd, hh, ww), loss[0, 0], uniq[0, 0])


# speedup vs baseline: 1.2136x; 1.2136x over previous
import functools

import jax
import jax.numpy as jnp
from jax.experimental import pallas as pl
from jax.experimental.pallas import tpu as pltpu

_NUM_EMBEDDINGS = 1024
_EMBEDDING_DIM = 64
_COMMITMENT_COST = 0.25


def _vq_block_kernel(zr_ref, w_ref, wm2_ref, zsq_ref, wsq_ref,
                     zq_ref, loss_ref, uniq_ref,
                     pres_ref, acc_ref, *, n_tokens, t_blk):
    i = pl.program_id(0)
    n_steps = pl.num_programs(0)

    @pl.when(i == 0)
    def _init():
        pres_ref[...] = jnp.zeros_like(pres_ref)
        acc_ref[0, 0] = jnp.float32(0.0)

    zr = zr_ref[0]
    w = w_ref[...]
    wm2 = wm2_ref[...]

    m2 = jax.lax.dot_general(wm2, zr, (((1,), (0,)), ((), ())),
                             preferred_element_type=jnp.float32)
    d = (zsq_ref[0] + m2) + wsq_ref[...]

    mind = jnp.min(d, axis=0, keepdims=True)
    iota = jax.lax.broadcasted_iota(jnp.int32, (_NUM_EMBEDDINGS, t_blk), 0)
    idx = jnp.min(jnp.where(d == mind, iota, _NUM_EMBEDDINGS),
                  axis=0).astype(jnp.int32)

    oh = (iota == idx[None, :]).astype(jnp.float32)
    zq_t = jax.lax.dot_general(w, oh, (((0,), (0,)), ((), ())),
                               preferred_element_type=jnp.float32)
    zq_ref[0] = zr + (zq_t - zr)

    pres_ref[...] = jnp.maximum(pres_ref[...], jnp.max(oh, axis=1,
                                                       keepdims=True))
    acc_ref[0, 0] += jnp.sum(mind[0])

    @pl.when(i == n_steps - 1)
    def _finish():
        total = acc_ref[0, 0]
        mean = total / jnp.float32(n_tokens * _EMBEDDING_DIM)
        loss_ref[0, 0] = mean + jnp.float32(_COMMITMENT_COST) * mean
        uniq_ref[0, 0] = jnp.sum(pres_ref[...]).astype(jnp.int32)


@jax.jit
def kernel(z, W):
    b, c, dd, hh, ww = z.shape
    n_tokens = b * dd * hh * ww
    spatial = dd * hh * ww
    t_blk = 512
    bb = spatial // t_blk
    z_raw = z.reshape(b, c, spatial)
    zsq = jnp.sum(z_raw ** 2, axis=1).reshape(b * bb, 1, t_blk)
    wsq = jnp.sum(W ** 2, axis=1).reshape(_NUM_EMBEDDINGS, 1)
    wm2 = -2.0 * W

    grid = (n_tokens // t_blk,)
    zq_raw, loss, uniq = pl.pallas_call(
        functools.partial(_vq_block_kernel, n_tokens=n_tokens, t_blk=t_blk),
        grid=grid,
        in_specs=[
            pl.BlockSpec((1, c, t_blk), lambda i: (i // bb, 0, i % bb)),
            pl.BlockSpec((_NUM_EMBEDDINGS, c), lambda i: (0, 0)),
            pl.BlockSpec((_NUM_EMBEDDINGS, c), lambda i: (0, 0)),
            pl.BlockSpec((1, 1, t_blk), lambda i: (i, 0, 0)),
            pl.BlockSpec((_NUM_EMBEDDINGS, 1), lambda i: (0, 0)),
        ],
        out_specs=[
            pl.BlockSpec((1, c, t_blk), lambda i: (i // bb, 0, i % bb)),
            pl.BlockSpec((1, 1), lambda i: (0, 0), memory_space=pltpu.SMEM),
            pl.BlockSpec((1, 1), lambda i: (0, 0), memory_space=pltpu.SMEM),
        ],
        out_shape=[
            jax.ShapeDtypeStruct((b, c, spatial), jnp.float32),
            jax.ShapeDtypeStruct((1, 1), jnp.float32),
            jax.ShapeDtypeStruct((1, 1), jnp.int32),
        ],
        scratch_shapes=[
            pltpu.VMEM((_NUM_EMBEDDINGS, 1), jnp.float32),
            pltpu.SMEM((1, 1), jnp.float32),
        ],
    )(z_raw, W, wm2, zsq, wsq)

    return (zq_raw.reshape(b, c, dd, hh, ww), loss[0, 0], uniq[0, 0])


# speedup vs baseline: 1.4017x; 1.1550x over previous
import functools

import jax
import jax.numpy as jnp
from jax.experimental import pallas as pl
from jax.experimental.pallas import tpu as pltpu

_NUM_EMBEDDINGS = 1024
_EMBEDDING_DIM = 64
_COMMITMENT_COST = 0.25


def _vq_block_kernel(zr_ref, w_ref, wm2_ref, zsq_ref, wsq_ref,
                     zq_ref, loss_ref, uniq_ref,
                     pres_ref, acc_ref, *, n_tokens, t_blk):
    i = pl.program_id(0)
    n_steps = pl.num_programs(0)

    @pl.when(i == 0)
    def _init():
        pres_ref[...] = jnp.zeros_like(pres_ref)
        acc_ref[0, 0] = jnp.float32(0.0)

    zr = zr_ref[0]
    w = w_ref[...]
    wm2 = wm2_ref[...]

    m2 = jax.lax.dot_general(wm2, zr, (((1,), (0,)), ((), ())),
                             preferred_element_type=jnp.float32)
    d = (zsq_ref[0] + m2) + wsq_ref[...]

    mind = jnp.min(d, axis=0, keepdims=True)
    iota = jax.lax.broadcasted_iota(jnp.int32, (_NUM_EMBEDDINGS, t_blk), 0)
    idx = jnp.min(jnp.where(d == mind, iota, _NUM_EMBEDDINGS),
                  axis=0).astype(jnp.int32)

    oh = (iota == idx[None, :]).astype(jnp.float32)
    zq_t = jax.lax.dot_general(w, oh, (((0,), (0,)), ((), ())),
                               preferred_element_type=jnp.float32)
    zq_ref[0] = zr + (zq_t - zr)

    pres_ref[...] = jnp.maximum(pres_ref[...], jnp.max(oh, axis=1,
                                                       keepdims=True))
    acc_ref[0, 0] += jnp.sum(mind[0])

    @pl.when(i == n_steps - 1)
    def _finish():
        total = acc_ref[0, 0]
        mean = total / jnp.float32(n_tokens * _EMBEDDING_DIM)
        loss_ref[0, 0] = mean + jnp.float32(_COMMITMENT_COST) * mean
        uniq_ref[0, 0] = jnp.sum(pres_ref[...]).astype(jnp.int32)


@jax.jit
def kernel(z, W):
    b, c, dd, hh, ww = z.shape
    n_tokens = b * dd * hh * ww
    spatial = dd * hh * ww
    t_blk = 1024
    bb = spatial // t_blk
    z_raw = z.reshape(b, c, spatial)
    zsq = jnp.sum(z_raw ** 2, axis=1).reshape(b * bb, 1, t_blk)
    wsq = jnp.sum(W ** 2, axis=1).reshape(_NUM_EMBEDDINGS, 1)
    wm2 = -2.0 * W

    grid = (n_tokens // t_blk,)
    zq_raw, loss, uniq = pl.pallas_call(
        functools.partial(_vq_block_kernel, n_tokens=n_tokens, t_blk=t_blk),
        grid=grid,
        in_specs=[
            pl.BlockSpec((1, c, t_blk), lambda i: (i // bb, 0, i % bb)),
            pl.BlockSpec((_NUM_EMBEDDINGS, c), lambda i: (0, 0)),
            pl.BlockSpec((_NUM_EMBEDDINGS, c), lambda i: (0, 0)),
            pl.BlockSpec((1, 1, t_blk), lambda i: (i, 0, 0)),
            pl.BlockSpec((_NUM_EMBEDDINGS, 1), lambda i: (0, 0)),
        ],
        out_specs=[
            pl.BlockSpec((1, c, t_blk), lambda i: (i // bb, 0, i % bb)),
            pl.BlockSpec((1, 1), lambda i: (0, 0), memory_space=pltpu.SMEM),
            pl.BlockSpec((1, 1), lambda i: (0, 0), memory_space=pltpu.SMEM),
        ],
        out_shape=[
            jax.ShapeDtypeStruct((b, c, spatial), jnp.float32),
            jax.ShapeDtypeStruct((1, 1), jnp.float32),
            jax.ShapeDtypeStruct((1, 1), jnp.int32),
        ],
        scratch_shapes=[
            pltpu.VMEM((_NUM_EMBEDDINGS, 1), jnp.float32),
            pltpu.SMEM((1, 1), jnp.float32),
        ],
    )(z_raw, W, wm2, zsq, wsq)

    return (zq_raw.reshape(b, c, dd, hh, ww), loss[0, 0], uniq[0, 0])


# speedup vs baseline: 1.4386x; 1.0263x over previous
import functools

import jax
import jax.numpy as jnp
from jax.experimental import pallas as pl
from jax.experimental.pallas import tpu as pltpu

_NUM_EMBEDDINGS = 1024
_EMBEDDING_DIM = 64
_COMMITMENT_COST = 0.25


def _vq_block_kernel(zr_ref, w_ref, wm2_ref, zsq_ref, wsq_ref,
                     zq_ref, loss_ref, uniq_ref,
                     pres_ref, acc_ref, *, n_tokens, t_blk):
    i = pl.program_id(0)
    n_steps = pl.num_programs(0)

    @pl.when(i == 0)
    def _init():
        pres_ref[...] = jnp.zeros_like(pres_ref)
        acc_ref[0, 0] = jnp.float32(0.0)

    zr = zr_ref[0]
    w = w_ref[...]
    wm2 = wm2_ref[...]

    m2 = jax.lax.dot_general(wm2, zr, (((1,), (0,)), ((), ())),
                             preferred_element_type=jnp.float32)
    d = (zsq_ref[0] + m2) + wsq_ref[...]

    mind = jnp.min(d, axis=0, keepdims=True)
    iota = jax.lax.broadcasted_iota(jnp.int32, (_NUM_EMBEDDINGS, t_blk), 0)
    idx = jnp.min(jnp.where(d == mind, iota, _NUM_EMBEDDINGS),
                  axis=0).astype(jnp.int32)

    oh = (iota == idx[None, :]).astype(jnp.float32)
    zq_t = jax.lax.dot_general(w, oh, (((0,), (0,)), ((), ())),
                               preferred_element_type=jnp.float32)
    zq_ref[0] = zr + (zq_t - zr)

    pres_ref[...] = jnp.maximum(pres_ref[...], jnp.max(oh, axis=1,
                                                       keepdims=True))
    acc_ref[0, 0] += jnp.sum(mind[0])

    @pl.when(i == n_steps - 1)
    def _finish():
        total = acc_ref[0, 0]
        mean = total / jnp.float32(n_tokens * _EMBEDDING_DIM)
        loss_ref[0, 0] = mean + jnp.float32(_COMMITMENT_COST) * mean
        uniq_ref[0, 0] = jnp.sum(pres_ref[...]).astype(jnp.int32)


@jax.jit
def kernel(z, W):
    b, c, dd, hh, ww = z.shape
    n_tokens = b * dd * hh * ww
    spatial = dd * hh * ww
    t_blk = 2048
    bb = spatial // t_blk
    z_raw = z.reshape(b, c, spatial)
    zsq = jnp.sum(z_raw ** 2, axis=1).reshape(b * bb, 1, t_blk)
    wsq = jnp.sum(W ** 2, axis=1).reshape(_NUM_EMBEDDINGS, 1)
    wm2 = -2.0 * W

    grid = (n_tokens // t_blk,)
    zq_raw, loss, uniq = pl.pallas_call(
        functools.partial(_vq_block_kernel, n_tokens=n_tokens, t_blk=t_blk),
        grid=grid,
        in_specs=[
            pl.BlockSpec((1, c, t_blk), lambda i: (i // bb, 0, i % bb)),
            pl.BlockSpec((_NUM_EMBEDDINGS, c), lambda i: (0, 0)),
            pl.BlockSpec((_NUM_EMBEDDINGS, c), lambda i: (0, 0)),
            pl.BlockSpec((1, 1, t_blk), lambda i: (i, 0, 0)),
            pl.BlockSpec((_NUM_EMBEDDINGS, 1), lambda i: (0, 0)),
        ],
        out_specs=[
            pl.BlockSpec((1, c, t_blk), lambda i: (i // bb, 0, i % bb)),
            pl.BlockSpec((1, 1), lambda i: (0, 0), memory_space=pltpu.SMEM),
            pl.BlockSpec((1, 1), lambda i: (0, 0), memory_space=pltpu.SMEM),
        ],
        out_shape=[
            jax.ShapeDtypeStruct((b, c, spatial), jnp.float32),
            jax.ShapeDtypeStruct((1, 1), jnp.float32),
            jax.ShapeDtypeStruct((1, 1), jnp.int32),
        ],
        scratch_shapes=[
            pltpu.VMEM((_NUM_EMBEDDINGS, 1), jnp.float32),
            pltpu.SMEM((1, 1), jnp.float32),
        ],
    )(z_raw, W, wm2, zsq, wsq)

    return (zq_raw.reshape(b, c, dd, hh, ww), loss[0, 0], uniq[0, 0])
